# Initial kernel scaffold; baseline (speedup 1.0000x reference)
import jax
import jax.numpy as jnp
from jax.experimental import pallas as pl


def kernel(xs, edge_indices, batches, train, params):
    raise NotImplementedError("write your pallas kernel here")


# speedup vs baseline: 1.0000x; 1.0000x over previous
import jax
import jax.numpy as jnp
from jax.experimental import pallas as pl

N = 10000
G = 64


def _gatv2(x, src, dst, p):
    xl = x @ p["Wl"]
    xr = x @ p["Wr"]
    e = jax.nn.leaky_relu(xl[src] + xr[dst], negative_slope=0.2)
    logits = e @ p["att"]
    m = jax.ops.segment_max(logits, dst, num_segments=N)
    ex = jnp.exp(logits - m[dst])
    denom = jax.ops.segment_sum(ex, dst, num_segments=N)
    alpha = ex / jnp.maximum(denom[dst], 1e-16)
    out = jax.ops.segment_sum(alpha[:, None] * xl[src], dst, num_segments=N)
    return out + p["b"]


def _batchnorm(x, p):
    mu = jnp.mean(x, axis=0)
    var = jnp.var(x, axis=0)
    return (x - mu) / jnp.sqrt(var + 1e-5) * p["g"] + p["b"]


def kernel(xs, edge_indices, batches, train, params):
    src = edge_indices[0]
    dst = edge_indices[1]
    h1 = jax.nn.relu(_gatv2(xs, src, dst, params["gat1"]))
    h1 = _batchnorm(h1, params["bn1"])
    h2 = jax.nn.relu(_gatv2(h1, src, dst, params["gat2"]))
    h2 = _batchnorm(h2, params["bn2"])
    h3 = jax.nn.relu(_gatv2(h2, src, dst, params["gat3"]))
    h3 = _batchnorm(h3, params["bn3"])
    h4 = h3
    p1 = jax.ops.segment_sum(h1, batches, num_segments=G)
    p2 = jax.ops.segment_sum(h2, batches, num_segments=G)
    p3 = jax.ops.segment_sum(h3, batches, num_segments=G)
    p4 = jax.ops.segment_sum(h4, batches, num_segments=G)
    h = jnp.concatenate((p1, p2, p3, p4), axis=1)
    h = jax.nn.relu(h @ params["lin1"]["W"] + params["lin1"]["b"])
    h = _batchnorm(h, params["bn5"])
    h = h @ params["lin2"]["W"] + params["lin2"]["b"]
    return (jax.nn.sigmoid(h), jax.nn.log_softmax(h, axis=1))


# speedup vs baseline: 2.7895x; 2.7894x over previous
import functools

import jax
import jax.numpy as jnp
from jax import lax
from jax.experimental import pallas as pl
from jax.experimental.pallas import tpu as pltpu
from jax.experimental.pallas import tpu_sc as plsc

N = 10000
E = 320000
G = 64
NC = 2
NS = 16
NTILES = NC * NS
EPT = E // NTILES
BLK = 80
NGRP = BLK // 16
NBLK = EPT // BLK
NEG = 0.2
BR = 1000
NRB = N // BR

_mesh = plsc.VectorSubcoreMesh(core_axis_name="c", subcore_axis_name="s",
                               num_cores=NC, num_subcores=NS)
_f32 = jnp.float32


@functools.lru_cache(maxsize=None)
def _sc_pass1(D):
    @functools.partial(
        pl.kernel,
        out_type=(
            jax.ShapeDtypeStruct((E,), _f32),
            jax.ShapeDtypeStruct((NC * N,), _f32),
        ),
        mesh=_mesh,
        scratch_types=[
            pltpu.VMEM((EPT,), jnp.int32),
            pltpu.VMEM((EPT,), jnp.int32),
            pltpu.VMEM((BLK,), jnp.int32),
            pltpu.VMEM((BLK, D), _f32),
            pltpu.VMEM((BLK, D), _f32),
            pltpu.VMEM((EPT,), _f32),
            pltpu.VMEM((D,), _f32),
            pltpu.VMEM_SHARED((N,), _f32),
            pltpu.SemaphoreType.DMA,
            pltpu.SemaphoreType.DMA,
        ],
        compiler_params=pltpu.CompilerParams(use_tc_tiling_on_sc=False, needs_layout_passes=False),
    )
    def k(xl_hbm, xr_hbm, att_hbm, src_hbm, dst_hbm, zden_hbm, ex_hbm,
          den_hbm, src_ids, dst_ids, dstb, xl_rows, xr_rows, ex_buf,
          att_v, denom_sp, sem1, sem2):
        cid = lax.axis_index("c")
        sid = lax.axis_index("s")
        wid = cid * NS + sid
        base = wid * EPT
        pltpu.sync_copy(src_hbm.at[pl.ds(base, EPT)], src_ids)
        pltpu.sync_copy(dst_hbm.at[pl.ds(base, EPT)], dst_ids)
        pltpu.sync_copy(att_hbm, att_v)

        @pl.when(sid < 10)
        def _():
            pltpu.sync_copy(zden_hbm, denom_sp.at[pl.ds(sid * 1000, 1000)])

        plsc.subcore_barrier()

        iota = lax.iota(jnp.int32, 16)

        def block(j, carry):
            off = j * BLK
            for g in range(NGRP):
                dstb[pl.ds(g * 16, 16)] = dst_ids[pl.ds(off + g * 16, 16)]
            c1 = pltpu.async_copy(xl_hbm.at[src_ids.at[pl.ds(off, BLK)]],
                                  xl_rows, sem1)
            c2 = pltpu.async_copy(xr_hbm.at[dst_ids.at[pl.ds(off, BLK)]],
                                  xr_rows, sem2)
            c1.wait()
            c2.wait()
            for g in range(NGRP):
                rows = g * 16 + iota

                def cstep(c, acc_cidx):
                    acc, cidx = acc_cidx
                    xlv = plsc.load_gather(xl_rows, [rows, cidx])
                    xrv = plsc.load_gather(xr_rows, [rows, cidx])
                    av = plsc.load_gather(att_v, [cidx])
                    s = xlv + xrv
                    lr = jnp.maximum(s, s * NEG)
                    return (acc + lr * av, cidx + 1)

                acc, _ = lax.fori_loop(
                    0, D, cstep,
                    (jnp.zeros((16,), _f32), jnp.zeros((16,), jnp.int32)))
                ex_buf[pl.ds(off + g * 16, 16)] = jnp.exp(acc)
            pltpu.sync_copy(ex_buf.at[pl.ds(off, BLK)],
                            denom_sp.at[dstb], add=True)
            return carry

        lax.fori_loop(0, NBLK, block, 0)
        plsc.subcore_barrier()

        pltpu.sync_copy(ex_buf, ex_hbm.at[pl.ds(base, EPT)])

        @pl.when(sid < 10)
        def _():
            pltpu.sync_copy(denom_sp.at[pl.ds(sid * 1000, 1000)],
                            den_hbm.at[pl.ds(cid * N + sid * 1000, 1000)])

    return k


@functools.lru_cache(maxsize=None)
def _sc_pass2(D):
    @functools.partial(
        pl.kernel,
        out_type=jax.ShapeDtypeStruct((NC, N, D), _f32),
        mesh=_mesh,
        scratch_types=[
            pltpu.VMEM((BLK,), jnp.int32),
            pltpu.VMEM((BLK,), jnp.int32),
            pltpu.VMEM((BLK,), _f32),
            pltpu.VMEM((N,), _f32),
            pltpu.VMEM((2000,), _f32),
            pltpu.VMEM((BLK, D), _f32),
            pltpu.VMEM((BLK, D), _f32),
            pltpu.VMEM_SHARED((N, D), _f32),
            pltpu.SemaphoreType.DMA,
        ],
        compiler_params=pltpu.CompilerParams(use_tc_tiling_on_sc=False, needs_layout_passes=False),
    )
    def k(xl_hbm, src_hbm, dst_hbm, ex_hbm, den_hbm, zrows_hbm, out_hbm,
          srcb, dstb, exb, den, tmp, xl_rows, sc_buf, out_sp, sem1):
        cid = lax.axis_index("c")
        sid = lax.axis_index("s")
        wid = cid * NS + sid
        base = wid * EPT
        pltpu.sync_copy(den_hbm.at[pl.ds(0, N)], den)

        def dchunk(i, c):
            pltpu.sync_copy(den_hbm.at[pl.ds(N + i * 2000, 2000)], tmp)

            def dadd(r, c2):
                sl = pl.ds(i * 2000 + r * 16, 16)
                den[sl] = den[sl] + tmp[pl.ds(r * 16, 16)]
                return c2
            lax.fori_loop(0, 125, dadd, 0)
            return c
        lax.fori_loop(0, 5, dchunk, 0)

        @pl.when(sid < 10)
        def _():
            pltpu.sync_copy(zrows_hbm, out_sp.at[pl.ds(sid * 1000, 1000)])

        plsc.subcore_barrier()

        iota = lax.iota(jnp.int32, 16)

        def block(j, carry):
            off = base + j * BLK
            pltpu.sync_copy(src_hbm.at[pl.ds(off, BLK)], srcb)
            pltpu.sync_copy(dst_hbm.at[pl.ds(off, BLK)], dstb)
            pltpu.sync_copy(ex_hbm.at[pl.ds(off, BLK)], exb)
            pltpu.async_copy(xl_hbm.at[srcb], xl_rows, sem1).wait()
            for g in range(NGRP):
                rows = g * 16 + iota
                ex16 = exb[pl.ds(g * 16, 16)]
                d16 = dstb[pl.ds(g * 16, 16)]
                denv = plsc.load_gather(den, [d16])
                alpha = ex16 / jnp.maximum(denv, 1e-16)

                def cstep(c, cidx):
                    v = plsc.load_gather(xl_rows, [rows, cidx])
                    plsc.store_scatter(sc_buf, [rows, cidx], v * alpha)
                    return cidx + 1

                lax.fori_loop(0, D, cstep, jnp.zeros((16,), jnp.int32))
            pltpu.sync_copy(sc_buf, out_sp.at[dstb], add=True)
            return carry

        lax.fori_loop(0, NBLK, block, 0)
        plsc.subcore_barrier()

        @pl.when(sid < 10)
        def _():
            pltpu.sync_copy(out_sp.at[pl.ds(sid * 1000, 1000)],
                            out_hbm.at[cid, pl.ds(sid * 1000, 1000)])

    return k


def _mm2(h, wl, wr):
    n, din = h.shape
    dout = wl.shape[1]

    def body(h_ref, wl_ref, wr_ref, xl_ref, xr_ref):
        hb = h_ref[...]
        xl_ref[...] = jnp.dot(hb, wl_ref[...], preferred_element_type=_f32)
        xr_ref[...] = jnp.dot(hb, wr_ref[...], preferred_element_type=_f32)

    return pl.pallas_call(
        body,
        grid=(n // BR,),
        in_specs=[
            pl.BlockSpec((BR, din), lambda i: (i, 0)),
            pl.BlockSpec((din, dout), lambda i: (0, 0)),
            pl.BlockSpec((din, dout), lambda i: (0, 0)),
        ],
        out_specs=[pl.BlockSpec((BR, dout), lambda i: (i, 0))] * 2,
        out_shape=[jax.ShapeDtypeStruct((n, dout), _f32)] * 2,
    )(h, wl, wr)


def _stats(outp, bvec):
    d = outp.shape[2]

    def body(o_ref, b_ref, h_ref, s_ref):
        i = pl.program_id(0)
        x = jnp.maximum(o_ref[0] + o_ref[1] + b_ref[...], 0.0)
        h_ref[...] = x

        @pl.when(i == 0)
        def _():
            s_ref[...] = jnp.zeros_like(s_ref)

        rs = jnp.sum(x, axis=0, keepdims=True)
        rq = jnp.sum(x * x, axis=0, keepdims=True)
        s_ref[...] += jnp.concatenate(
            [rs, rq, jnp.zeros((6, d), _f32)], axis=0)

    return pl.pallas_call(
        body,
        grid=(NRB,),
        in_specs=[
            pl.BlockSpec((2, BR, d), lambda i: (0, i, 0)),
            pl.BlockSpec((1, d), lambda i: (0, 0)),
        ],
        out_specs=[
            pl.BlockSpec((BR, d), lambda i: (i, 0)),
            pl.BlockSpec((8, d), lambda i: (0, 0)),
        ],
        out_shape=[
            jax.ShapeDtypeStruct((N, d), _f32),
            jax.ShapeDtypeStruct((8, d), _f32),
        ],
    )(outp, bvec)


def _apply_pool_mm(h_pre, sums, gb, batches3, wl=None, wr=None):
    d = h_pre.shape[1]
    have_mm = wl is not None

    def body(*refs):
        if have_mm:
            (hp_ref, s_ref, gb_ref, b3_ref, wl_ref, wr_ref,
             xl_ref, xr_ref, p_ref) = refs
        else:
            hp_ref, s_ref, gb_ref, b3_ref, p_ref = refs
        i = pl.program_id(0)
        mu = s_ref[0:1, :] / N
        var = s_ref[1:2, :] / N - mu * mu
        inv = lax.rsqrt(var + 1e-5)
        x = (hp_ref[...] - mu) * inv * gb_ref[0:1, :] + gb_ref[1:2, :]
        if have_mm:
            xl_ref[...] = jnp.dot(x, wl_ref[...], preferred_element_type=_f32)
            xr_ref[...] = jnp.dot(x, wr_ref[...], preferred_element_type=_f32)
        bb = b3_ref[0, 0, :]
        oh = (bb[:, None] == lax.broadcasted_iota(jnp.int32, (BR, G), 1)
              ).astype(_f32)
        pool = lax.dot_general(oh, x, (((0,), (0,)), ((), ())),
                               preferred_element_type=_f32)

        @pl.when(i == 0)
        def _():
            p_ref[...] = jnp.zeros_like(p_ref)

        p_ref[...] += pool

    in_specs = [
        pl.BlockSpec((BR, d), lambda i: (i, 0)),
        pl.BlockSpec((8, d), lambda i: (0, 0)),
        pl.BlockSpec((2, d), lambda i: (0, 0)),
        pl.BlockSpec((1, 1, BR), lambda i: (i, 0, 0)),
    ]
    out_specs = [pl.BlockSpec((G, d), lambda i: (0, 0))]
    out_shape = [jax.ShapeDtypeStruct((G, d), _f32)]
    args = [h_pre, sums, gb, batches3]
    if have_mm:
        dout = wl.shape[1]
        in_specs += [pl.BlockSpec((d, dout), lambda i: (0, 0))] * 2
        out_specs = ([pl.BlockSpec((BR, dout), lambda i: (i, 0))] * 2
                     + out_specs)
        out_shape = ([jax.ShapeDtypeStruct((N, dout), _f32)] * 2 + out_shape)
        args += [wl, wr]

    return pl.pallas_call(
        body,
        grid=(NRB,),
        in_specs=in_specs,
        out_specs=out_specs,
        out_shape=out_shape,
    )(*args)


def _head(p1, p2, p3, w1, b1, gb5, w2, b2):
    def body(p1_ref, p2_ref, p3_ref, w1_ref, b1_ref, gb_ref, w2_ref, b2_ref,
             sig_ref, lsm_ref):
        h = jnp.concatenate(
            [p1_ref[...], p2_ref[...], p3_ref[...], p3_ref[...]], axis=1)
        h = jnp.maximum(
            jnp.dot(h, w1_ref[...], preferred_element_type=_f32)
            + b1_ref[...], 0.0)
        mu = jnp.mean(h, axis=0, keepdims=True)
        var = jnp.mean(h * h, axis=0, keepdims=True) - mu * mu
        h = (h - mu) * lax.rsqrt(var + 1e-5) * gb_ref[0:1, :] + gb_ref[1:2, :]
        lo = jnp.dot(h, w2_ref[...], preferred_element_type=_f32) + b2_ref[...]
        sig_ref[...] = jax.nn.sigmoid(lo)
        m = jnp.max(lo, axis=1, keepdims=True)
        lse = m + jnp.log(jnp.sum(jnp.exp(lo - m), axis=1, keepdims=True))
        lsm_ref[...] = lo - lse

    return pl.pallas_call(
        body,
        out_shape=[
            jax.ShapeDtypeStruct((G, 10), _f32),
            jax.ShapeDtypeStruct((G, 10), _f32),
        ],
    )(p1, p2, p3, w1, b1, gb5, w2, b2)


def _gat_layer(h, src, dst, p):
    d = p["Wl"].shape[1]
    xl, xr = _mm2(h, p["Wl"], p["Wr"])
    zden = jnp.zeros((1000,), _f32)
    zrows = jnp.zeros((1000, d), _f32)
    ex, den = _sc_pass1(d)(xl, xr, p["att"], src, dst, zden)
    outp = _sc_pass2(d)(xl, src, dst, ex, den, zrows)
    return _stats(outp, p["b"][None])


def kernel(xs, edge_indices, batches, train, params):
    src = edge_indices[0]
    dst = edge_indices[1]
    batches3 = batches.reshape(NRB, 1, BR)

    h1p, s1 = _gat_layer(xs, src, dst, params["gat1"])
    gb1 = jnp.stack([params["bn1"]["g"], params["bn1"]["b"]])
    xl2h, xr2h, p1 = _apply_pool_mm(h1p, s1, gb1, batches3,
                                    params["gat2"]["Wl"], params["gat2"]["Wr"])

    d2 = params["gat2"]["Wl"].shape[1]
    ex2, den2 = _sc_pass1(d2)(xl2h, xr2h, params["gat2"]["att"], src, dst,
                              jnp.zeros((1000,), _f32))
    out2 = _sc_pass2(d2)(xl2h, src, dst, ex2, den2, jnp.zeros((1000, d2), _f32))
    h2p, s2 = _stats(out2, params["gat2"]["b"][None])
    gb2 = jnp.stack([params["bn2"]["g"], params["bn2"]["b"]])
    xl3h, xr3h, p2 = _apply_pool_mm(h2p, s2, gb2, batches3,
                                    params["gat3"]["Wl"], params["gat3"]["Wr"])

    d3 = params["gat3"]["Wl"].shape[1]
    ex3, den3 = _sc_pass1(d3)(xl3h, xr3h, params["gat3"]["att"], src, dst,
                              jnp.zeros((1000,), _f32))
    out3 = _sc_pass2(d3)(xl3h, src, dst, ex3, den3, jnp.zeros((1000, d3), _f32))
    h3p, s3 = _stats(out3, params["gat3"]["b"][None])
    gb3 = jnp.stack([params["bn3"]["g"], params["bn3"]["b"]])
    p3 = _apply_pool_mm(h3p, s3, gb3, batches3)[0]

    gb5 = jnp.stack([params["bn5"]["g"], params["bn5"]["b"]])
    sig, lsm = _head(p1, p2, p3,
                     params["lin1"]["W"], params["lin1"]["b"][None], gb5,
                     params["lin2"]["W"], params["lin2"]["b"][None])
    return (sig, lsm)
